# baseline (speedup 1.0000x reference)
import functools

import jax
import jax.numpy as jnp
import numpy as np
from jax import lax
from jax.experimental import pallas as pl
from jax.experimental.pallas import tpu as pltpu
from jax.experimental.pallas import tpu_sc as plsc

B, T, D = 64, 8192, 16
N = B * T
NC, NS = 2, 16
NW = NC * NS
TPW = N // NW
CHUNK = 2048
NCHUNK = TPW // CHUNK
G = 16

MAGIC = np.float32(12582912.0)

_S = [np.float32(-1.0), np.float32(0.0), np.float32(-5.0)]
_E = [np.float32(1.0), np.float32(10.0), np.float32(5.0)]
_SZ = [256.0, 1024.0, 512.0]
_R = [np.float32(1.0) / (np.float32(e - s) / np.float32(sz - 1.0))
      for s, e, sz in zip(_S, _E, _SZ)]

_mesh = plsc.VectorSubcoreMesh(
    core_axis_name="c", subcore_axis_name="s", num_cores=NC, num_subcores=NS)


@functools.partial(
    pl.kernel,
    mesh=_mesh,
    out_type=jax.ShapeDtypeStruct((N * D,), jnp.float32),
    scratch_types=[
        pltpu.VMEM((CHUNK * D,), jnp.float32),
        pltpu.VMEM((CHUNK,), jnp.int32),
    ],
    compiler_params=pltpu.CompilerParams(needs_layout_passes=False),
)
def _encode(tks_hbm, ids_hbm, out_hbm, xv, iv):
    wid = lax.axis_index("s") * NC + lax.axis_index("c")
    base0 = wid * TPW
    lanes = lax.iota(jnp.int32, G)

    def chunk_body(ci, carry):
        base = base0 + ci * CHUNK
        pltpu.sync_copy(tks_hbm.at[pl.ds(base * D, CHUNK * D)], xv)
        pltpu.sync_copy(ids_hbm.at[pl.ds(base, CHUNK)], iv)

        def group_body(g, carry2):
            addr0 = (g * G + lanes) * D
            ids16 = iv[pl.ds(g * G, G)]
            c0 = ids16 == 0
            c1 = ids16 == 1
            c2 = ids16 == 2
            s16 = jnp.where(c0, _S[0], jnp.where(c1, _S[1], _S[2]))
            e16 = jnp.where(c0, _E[0], jnp.where(c1, _E[1], _E[2]))
            r16 = jnp.where(c0, _R[0], jnp.where(c1, _R[1], _R[2]))
            keep = ids16 == 3
            for c in range(D):
                addr = addr0 + c
                x = plsc.load_gather(xv, [addr])
                q = (jnp.minimum(jnp.maximum(x, s16), e16) - s16) * r16
                q = (q + MAGIC) - MAGIC
                plsc.store_scatter(xv, [addr], jnp.where(keep, x, q))
            return carry2

        lax.fori_loop(0, CHUNK // G, group_body, 0)
        pltpu.sync_copy(xv, out_hbm.at[pl.ds(base * D, CHUNK * D)])
        return carry

    lax.fori_loop(0, NCHUNK, chunk_body, 0)


def kernel(tks, tk_ids):
    out = _encode(tks.astype(jnp.float32).reshape(N * D), tk_ids.reshape(N))
    return out.reshape(B, T, D)


# speedup vs baseline: 3.1767x; 3.1767x over previous
import functools

import jax
import jax.numpy as jnp
import numpy as np
from jax import lax
from jax.experimental import pallas as pl
from jax.experimental.pallas import tpu as pltpu
from jax.experimental.pallas import tpu_sc as plsc

B, T, D = 64, 8192, 16
R = B * D
NC, NS = 2, 16
NW = NC * NS
TB = T // NW
RB = 64
NBLK = R // RB
G = 16

MAGIC = np.float32(12582912.0)

_S = [np.float32(-1.0), np.float32(0.0), np.float32(-5.0)]
_E = [np.float32(1.0), np.float32(10.0), np.float32(5.0)]
_SZ = [256.0, 1024.0, 512.0]
_IR = [np.float32(1.0) / (np.float32(e - s) / np.float32(sz - 1.0))
       for s, e, sz in zip(_S, _E, _SZ)]

_mesh = plsc.VectorSubcoreMesh(
    core_axis_name="c", subcore_axis_name="s", num_cores=NC, num_subcores=NS)


@functools.partial(
    pl.kernel,
    mesh=_mesh,
    out_type=jax.ShapeDtypeStruct((R, T), jnp.float32),
    scratch_types=[
        pltpu.VMEM((B, TB), jnp.int32),
        pltpu.VMEM((RB, TB), jnp.float32),
        pltpu.VMEM((RB, TB), jnp.float32),
    ],
    compiler_params=pltpu.CompilerParams(
        needs_layout_passes=False, use_tc_tiling_on_sc=True),
)
def _encode(x_hbm, ids_hbm, out_hbm, idv, xv, ov):
    wid = lax.axis_index("s") * NC + lax.axis_index("c")
    col0 = wid * TB
    pltpu.sync_copy(ids_hbm.at[:, pl.ds(col0, TB)], idv)

    def blk_body(blk, carry):
        pltpu.sync_copy(x_hbm.at[pl.ds(blk * RB, RB), pl.ds(col0, TB)], xv)

        def bb_body(bb, carry2):
            def g_body(g, carry3):
                ids16 = idv[blk * (RB // D) + bb, pl.ds(g * G, G)]
                c0 = ids16 == 0
                c1 = ids16 == 1
                s16 = jnp.where(c0, _S[0], jnp.where(c1, _S[1], _S[2]))
                e16 = jnp.where(c0, _E[0], jnp.where(c1, _E[1], _E[2]))
                r16 = jnp.where(c0, _IR[0], jnp.where(c1, _IR[1], _IR[2]))
                keep = ids16 == 3
                for c in range(D):
                    x = xv[bb * D + c, pl.ds(g * G, G)]
                    q = (jnp.minimum(jnp.maximum(x, s16), e16) - s16) * r16
                    q = (q + MAGIC) - MAGIC
                    ov[bb * D + c, pl.ds(g * G, G)] = jnp.where(keep, x, q)
                return carry3

            return lax.fori_loop(0, TB // G, g_body, carry2)

        lax.fori_loop(0, RB // D, bb_body, 0)
        pltpu.sync_copy(ov, out_hbm.at[pl.ds(blk * RB, RB), pl.ds(col0, TB)])
        return carry

    lax.fori_loop(0, NBLK, blk_body, 0)


def kernel(tks, tk_ids):
    xt = jnp.transpose(tks.astype(jnp.float32), (0, 2, 1)).reshape(R, T)
    out = _encode(xt, tk_ids)
    return jnp.transpose(out.reshape(B, D, T), (0, 2, 1))


# speedup vs baseline: 3.6531x; 1.1500x over previous
import functools

import jax
import jax.numpy as jnp
import numpy as np
from jax import lax
from jax.experimental import pallas as pl
from jax.experimental.pallas import tpu as pltpu
from jax.experimental.pallas import tpu_sc as plsc

B, T, D = 64, 8192, 16
R = B * D
NC, NS = 2, 16
NW = NC * NS
TB = T // NW
RB = 64
NBLK = R // RB
G = 16

MAGIC = np.float32(12582912.0)

_S = [-1.0, 0.0, -5.0, 0.0]
_E = [1.0, 10.0, 5.0, 1.0]
_SZ = [256.0, 1024.0, 512.0, 2.0]
_IR = [float(np.float32(1.0) / (np.float32(e - s) / np.float32(sz - 1.0)))
       for s, e, sz in zip(_S, _E, _SZ)]
_S_TAB = tuple(_S + [0.0] * 12)
_E_TAB = tuple(_E + [1.0] * 12)
_IR_TAB = tuple(_IR + [1.0] * 12)

_mesh = plsc.VectorSubcoreMesh(
    core_axis_name="c", subcore_axis_name="s", num_cores=NC, num_subcores=NS)


@functools.partial(
    pl.kernel,
    mesh=_mesh,
    out_type=jax.ShapeDtypeStruct((R, T), jnp.float32),
    scratch_types=[
        pltpu.VMEM((B, TB), jnp.int32),
        pltpu.VMEM((RB, TB), jnp.float32),
        pltpu.VMEM((RB, TB), jnp.float32),
        pltpu.VMEM((RB, TB), jnp.float32),
        pltpu.VMEM((RB, TB), jnp.float32),
        pltpu.VMEM((3, G), jnp.float32),
        pltpu.SemaphoreType.DMA,
        pltpu.SemaphoreType.DMA,
        pltpu.SemaphoreType.DMA,
        pltpu.SemaphoreType.DMA,
    ],
    compiler_params=pltpu.CompilerParams(
        needs_layout_passes=False, use_tc_tiling_on_sc=True),
)
def _encode(x_hbm, ids_hbm, tabs_hbm, out_hbm, idv, xv0, xv1, ov0, ov1,
            tabv, si0, si1, so0, so1):
    wid = lax.axis_index("s") * NC + lax.axis_index("c")
    col0 = wid * TB
    pltpu.sync_copy(tabs_hbm, tabv)
    pltpu.sync_copy(ids_hbm.at[:, pl.ds(col0, TB)], idv)

    s_tab = tabv[0, :]
    e_tab = tabv[1, :]
    ir_tab = tabv[2, :]

    def in_copy(blk, buf, sem):
        return pltpu.make_async_copy(
            x_hbm.at[pl.ds(blk * RB, RB), pl.ds(col0, TB)], buf, sem)

    def out_copy(blk, buf, sem):
        return pltpu.make_async_copy(
            buf, out_hbm.at[pl.ds(blk * RB, RB), pl.ds(col0, TB)], sem)

    def compute(blk, xvb, ovb):
        def bb_body(bb, carry2):
            def g_body(g, carry3):
                ids16 = idv[blk * (RB // D) + bb, pl.ds(g * G, G)]
                s16 = s_tab.at[ids16].get(mode="promise_in_bounds")
                e16 = e_tab.at[ids16].get(mode="promise_in_bounds")
                r16 = ir_tab.at[ids16].get(mode="promise_in_bounds")
                keep = ids16 == 3
                for c in range(D):
                    x = xvb[bb * D + c, pl.ds(g * G, G)]
                    q = (jnp.minimum(jnp.maximum(x, s16), e16) - s16) * r16
                    q = (q + MAGIC) - MAGIC
                    ovb[bb * D + c, pl.ds(g * G, G)] = jnp.where(keep, x, q)
                return carry3

            return lax.fori_loop(0, TB // G, g_body, carry2)

        lax.fori_loop(0, RB // D, bb_body, 0)

    in_copy(0, xv0, si0).start()
    in_copy(1, xv1, si1).start()

    def pair_body(p, carry):
        blk0 = 2 * p
        blk1 = 2 * p + 1

        @pl.when(p > 0)
        def _():
            out_copy(blk0, ov0, so0).wait()

        in_copy(blk0, xv0, si0).wait()
        compute(blk0, xv0, ov0)
        out_copy(blk0, ov0, so0).start()

        @pl.when(blk0 + 2 < NBLK)
        def _():
            in_copy(blk0 + 2, xv0, si0).start()

        @pl.when(p > 0)
        def _():
            out_copy(blk1, ov1, so1).wait()

        in_copy(blk1, xv1, si1).wait()
        compute(blk1, xv1, ov1)
        out_copy(blk1, ov1, so1).start()

        @pl.when(blk1 + 2 < NBLK)
        def _():
            in_copy(blk1 + 2, xv1, si1).start()

        return carry

    lax.fori_loop(0, NBLK // 2, pair_body, 0)
    out_copy(NBLK - 2, ov0, so0).wait()
    out_copy(NBLK - 1, ov1, so1).wait()


_TABS = np.stack([_S_TAB, _E_TAB, _IR_TAB]).astype(np.float32)


def kernel(tks, tk_ids):
    xt = jnp.transpose(tks.astype(jnp.float32), (0, 2, 1)).reshape(R, T)
    out = _encode(xt, tk_ids, jnp.asarray(_TABS))
    return jnp.transpose(out.reshape(B, D, T), (0, 2, 1))


# speedup vs baseline: 12.1428x; 3.3239x over previous
import functools

import jax
import jax.numpy as jnp
import numpy as np
from jax import lax
from jax.experimental import pallas as pl
from jax.experimental.pallas import tpu as pltpu
from jax.experimental.pallas import tpu_sc as plsc

B, T, D = 64, 8192, 16
R = B * D
NC, NS = 2, 16
NW = NC * NS
TB = T // NW
RB = 64
NBLK = R // RB
G = 16

MAGIC = np.float32(12582912.0)

_S = [-1.0, 0.0, -5.0, 0.0]
_E = [1.0, 10.0, 5.0, 1.0]
_SZ = [256.0, 1024.0, 512.0, 2.0]
_IR = [float(np.float32(1.0) / (np.float32(e - s) / np.float32(sz - 1.0)))
       for s, e, sz in zip(_S, _E, _SZ)]
_S_TAB = tuple(_S + [0.0] * 12)
_E_TAB = tuple(_E + [1.0] * 12)
_IR_TAB = tuple(_IR + [1.0] * 12)

_mesh = plsc.VectorSubcoreMesh(
    core_axis_name="c", subcore_axis_name="s", num_cores=NC, num_subcores=NS)


@functools.partial(
    pl.kernel,
    mesh=_mesh,
    out_type=jax.ShapeDtypeStruct((R, T), jnp.float32),
    scratch_types=[
        pltpu.VMEM((B, TB), jnp.int32),
        pltpu.VMEM((RB, TB), jnp.float32),
        pltpu.VMEM((RB, TB), jnp.float32),
        pltpu.VMEM((RB, TB), jnp.float32),
        pltpu.VMEM((RB, TB), jnp.float32),
        pltpu.VMEM((3, G), jnp.float32),
        pltpu.SemaphoreType.DMA,
        pltpu.SemaphoreType.DMA,
        pltpu.SemaphoreType.DMA,
        pltpu.SemaphoreType.DMA,
    ],
    compiler_params=pltpu.CompilerParams(
        needs_layout_passes=False, use_tc_tiling_on_sc=True),
)
def _encode(x_hbm, ids_hbm, tabs_hbm, out_hbm, idv, xv0, xv1, ov0, ov1,
            tabv, si0, si1, so0, so1):
    wid = lax.axis_index("s") * NC + lax.axis_index("c")
    col0 = wid * TB
    pltpu.sync_copy(tabs_hbm, tabv)
    pltpu.sync_copy(ids_hbm.at[:, pl.ds(col0, TB)], idv)

    s_tab = tabv[0, :]
    e_tab = tabv[1, :]
    ir_tab = tabv[2, :]

    def in_copy(blk, buf, sem):
        return pltpu.make_async_copy(
            x_hbm.at[pl.ds(blk * RB, RB), pl.ds(col0, TB)], buf, sem)

    def out_copy(blk, buf, sem):
        return pltpu.make_async_copy(
            buf, out_hbm.at[pl.ds(blk * RB, RB), pl.ds(col0, TB)], sem)

    def compute(blk, xvb, ovb):
        def bb_body(bb, carry2):
            def g_body(g, carry3):
                ids16 = idv[blk * (RB // D) + bb, pl.ds(g * G, G)]
                s16 = s_tab.at[ids16].get(mode="promise_in_bounds")
                e16 = e_tab.at[ids16].get(mode="promise_in_bounds")
                r16 = ir_tab.at[ids16].get(mode="promise_in_bounds")
                keep = ids16 == 3
                xs = [xvb[bb * D + c, pl.ds(g * G, G)] for c in range(D)]
                q = [jnp.maximum(x, s16) for x in xs]
                q = [jnp.minimum(v, e16) for v in q]
                q = [v - s16 for v in q]
                q = [v * r16 for v in q]
                q = [v + MAGIC for v in q]
                q = [v - MAGIC for v in q]
                q = [jnp.where(keep, x, v) for x, v in zip(xs, q)]
                for c in range(D):
                    ovb[bb * D + c, pl.ds(g * G, G)] = q[c]
                return carry3

            return lax.fori_loop(0, TB // G, g_body, carry2)

        lax.fori_loop(0, RB // D, bb_body, 0)

    in_copy(0, xv0, si0).start()
    in_copy(1, xv1, si1).start()

    def pair_body(p, carry):
        blk0 = 2 * p
        blk1 = 2 * p + 1

        @pl.when(p > 0)
        def _():
            out_copy(blk0, ov0, so0).wait()

        in_copy(blk0, xv0, si0).wait()
        compute(blk0, xv0, ov0)
        out_copy(blk0, ov0, so0).start()

        @pl.when(blk0 + 2 < NBLK)
        def _():
            in_copy(blk0 + 2, xv0, si0).start()

        @pl.when(p > 0)
        def _():
            out_copy(blk1, ov1, so1).wait()

        in_copy(blk1, xv1, si1).wait()
        compute(blk1, xv1, ov1)
        out_copy(blk1, ov1, so1).start()

        @pl.when(blk1 + 2 < NBLK)
        def _():
            in_copy(blk1 + 2, xv1, si1).start()

        return carry

    lax.fori_loop(0, NBLK // 2, pair_body, 0)
    out_copy(NBLK - 2, ov0, so0).wait()
    out_copy(NBLK - 1, ov1, so1).wait()


_TABS = np.stack([_S_TAB, _E_TAB, _IR_TAB]).astype(np.float32)


def kernel(tks, tk_ids):
    xt = jnp.transpose(tks.astype(jnp.float32), (0, 2, 1)).reshape(R, T)
    out = _encode(xt, tk_ids, jnp.asarray(_TABS))
    return jnp.transpose(out.reshape(B, D, T), (0, 2, 1))
